# baseline (speedup 1.0000x reference)
import functools

import jax
import jax.numpy as jnp
from jax import lax
from jax.experimental import pallas as pl
from jax.experimental.pallas import tpu as pltpu, tpu_sc as plsc

N = 10000
E = 320000
D_IN = 128
D_H = 256
N_CLS = 47

NP = 10240
ROWS_PER = NP // 16
B = 128
EPC_E = 80 * B
EPC_C = 160 * B
EP = 32 * EPC_E


DROWS = NP // 128


def _make_agg(mode: str, with_deg: bool = False):
    s_type = jax.ShapeDtypeStruct((2, NP, 128), jnp.float32)
    out_type = [s_type,
                jax.ShapeDtypeStruct((32 * DROWS, 128), jnp.float32)] \
        if with_deg else s_type
    scratch = [
        pltpu.VMEM((B,), jnp.int32),
        pltpu.VMEM((B,), jnp.int32),
        pltpu.VMEM((B, 128), jnp.float32),
        pltpu.VMEM_SHARED((NP, 128), jnp.float32),
        pltpu.SemaphoreType.DMA,
    ]
    if with_deg:
        scratch.append(pltpu.VMEM((DROWS, 128), jnp.float32))
    mesh = plsc.VectorSubcoreMesh(core_axis_name="c", subcore_axis_name="s")

    @functools.partial(
        pl.kernel, mesh=mesh, out_type=out_type, scratch_types=scratch,
        compiler_params=pltpu.CompilerParams(
            needs_layout_passes=not with_deg))
    def agg(*refs):
        dacc = None
        if with_deg:
            dacc = refs[-1]
            refs = refs[:-1]
        if mode == "edge":
            table, src_hbm, dst_hbm, zrows, s_out = refs[:5]
            deg_out = refs[5] if with_deg else None
        else:
            table0, table1, src_hbm, dst_hbm, zrows, s_out = refs[:6]
        src_v, dst_v, rows_v, acc, sem = refs[-5:]

        cid = lax.axis_index("c")
        sid = lax.axis_index("s")
        r0 = sid * ROWS_PER

        def sweep(table, e0, nchunk):
            def step(i, carry):
                off = (e0 + i) * B
                pltpu.sync_copy(src_hbm.at[pl.ds(off, B)], src_v)
                pltpu.sync_copy(dst_hbm.at[pl.ds(off, B)], dst_v)
                desc = pltpu.async_copy(table.at[src_v], rows_v, sem)
                if with_deg:
                    ones16 = jnp.ones((16,), jnp.float32)
                    for k in range(B // 16):
                        d = dst_v[pl.ds(k * 16, 16)]
                        drow = lax.shift_right_logical(d, 7)
                        dcol = lax.bitwise_and(d, 127)
                        plsc.addupdate_scatter(dacc, [drow, dcol], ones16)
                desc.wait()
                pltpu.sync_copy(rows_v, acc.at[dst_v], add=True)
                return carry
            lax.fori_loop(0, nchunk, step, 0)

        def run(table, out_idx):
            pltpu.sync_copy(zrows.at[pl.ds(r0, ROWS_PER)],
                            acc.at[pl.ds(r0, ROWS_PER)])
            if with_deg:
                def zd(i, c):
                    for k in range(8):
                        dacc[i, pl.ds(k * 16, 16)] = jnp.zeros(
                            (16,), jnp.float32)
                    return c
                lax.fori_loop(0, DROWS, zd, 0)
            plsc.subcore_barrier()
            if mode == "chan":
                sweep(table, sid * (EPC_C // B), EPC_C // B)
            else:
                wid = sid * 2 + out_idx
                sweep(table, wid * (EPC_E // B), EPC_E // B)
                if with_deg:
                    pltpu.sync_copy(dacc,
                                    deg_out.at[pl.ds(wid * DROWS, DROWS)])
            plsc.subcore_barrier()
            pltpu.sync_copy(acc.at[pl.ds(r0, ROWS_PER)],
                            s_out.at[out_idx, pl.ds(r0, ROWS_PER)])

        if mode == "chan":
            @pl.when(cid == 0)
            def _():
                run(table0, 0)

            @pl.when(cid == 1)
            def _():
                run(table1, 1)
        else:
            @pl.when(cid == 0)
            def _():
                run(table, 0)

            @pl.when(cid == 1)
            def _():
                run(table, 1)

    return agg


def _dot(a, b):
    return jnp.dot(a, b, preferred_element_type=jnp.float32)


BM = 512


def _inv_deg(deg_ref):
    return 1.0 / jnp.maximum(deg_ref[:, 0:1], 1.0)


def _tc_degsum(dp_ref, out_ref):
    acc = dp_ref[0]
    for w in range(1, 32):
        acc = acc + dp_ref[w]
    out_ref[...] = acc


def _tc_a(x_ref, s0_ref, degp_ref, ws_ref, wn_ref, b_ref, out_ref):
    neigh = (s0_ref[0] + s0_ref[1]) * _inv_deg(degp_ref)
    h = _dot(x_ref[...], ws_ref[...]) + _dot(neigh, wn_ref[...]) + b_ref[...]
    h = jnp.maximum(h, 0.0)
    out_ref[0] = h[:, :D_H // 2]
    out_ref[1] = h[:, D_H // 2:]


def _tc_b(h1_ref, s1_ref, degp_ref, ws_ref, wn_ref, b_ref, wn2_ref,
          h2_ref, p2_ref):
    h1 = jnp.concatenate([h1_ref[0], h1_ref[1]], axis=1)
    neigh = jnp.concatenate([s1_ref[0], s1_ref[1]], axis=1) * _inv_deg(degp_ref)
    h2 = _dot(h1, ws_ref[...]) + _dot(neigh, wn_ref[...]) + b_ref[...]
    h2 = jnp.maximum(h2, 0.0)
    h2_ref[...] = h2
    p2_ref[...] = _dot(h2, wn2_ref[...])


def _tc_c(h2_ref, s2_ref, degp_ref, ws_ref, b_ref, out_ref):
    neigh = (s2_ref[0] + s2_ref[1]) * _inv_deg(degp_ref)
    out_ref[...] = _dot(h2_ref[...], ws_ref[...]) + neigh + b_ref[...]


def _row_spec(d):
    return pl.BlockSpec((BM, d), lambda i: (i, 0))


def _half_spec(d):
    return pl.BlockSpec((2, BM, d), lambda i: (0, i, 0))


def _full_spec(shape):
    return pl.BlockSpec(shape, lambda i: tuple(0 for _ in shape))


def kernel(x, edge_index, W_self0, W_neigh0, b0, W_self1, W_neigh1, b1,
           W_self2, W_neigh2, b2):
    f32 = jnp.float32
    src = edge_index[0]
    dst = edge_index[1]
    src_p = jnp.concatenate([src, jnp.zeros((EP - E,), jnp.int32)])
    dst_p = jnp.concatenate([dst, jnp.full((EP - E,), N, jnp.int32)])

    x_p = jnp.pad(x, ((0, NP - N), (0, 0)))
    z128 = jnp.zeros((NP, 128), f32)

    wn2p = jnp.pad(W_neigh2, ((0, 0), (0, 128 - N_CLS)))
    ws2p = jnp.pad(W_self2, ((0, 0), (0, 128 - N_CLS)))
    b0r = b0.reshape(1, D_H)
    b1r = b1.reshape(1, D_H)
    b2r = jnp.pad(b2, (0, 128 - N_CLS)).reshape(1, 128)

    s0, degq = _make_agg("edge", with_deg=True)(x_p, src_p, dst_p, z128)

    degsum = pl.pallas_call(
        _tc_degsum,
        grid=(1,),
        in_specs=[_full_spec((32, DROWS, 128))],
        out_specs=_full_spec((DROWS, 128)),
        out_shape=jax.ShapeDtypeStruct((DROWS, 128), f32),
    )(degq.reshape(32, DROWS, 128))
    deg16 = jnp.broadcast_to(degsum.reshape(NP, 1), (NP, 16))

    grid = (NP // BM,)
    h1h = pl.pallas_call(
        _tc_a,
        grid=grid,
        in_specs=[_row_spec(D_IN), _half_spec(128), _row_spec(16),
                  _full_spec((D_IN, D_H)), _full_spec((D_IN, D_H)),
                  _full_spec((1, D_H))],
        out_specs=_half_spec(D_H // 2),
        out_shape=jax.ShapeDtypeStruct((2, NP, D_H // 2), f32),
    )(x_p, s0, deg16, W_self0, W_neigh0, b0r)

    s1 = _make_agg("chan")(h1h[0], h1h[1], src_p, dst_p, z128)

    h2, p2p = pl.pallas_call(
        _tc_b,
        grid=grid,
        in_specs=[_half_spec(D_H // 2), _half_spec(D_H // 2), _row_spec(16),
                  _full_spec((D_H, D_H)), _full_spec((D_H, D_H)),
                  _full_spec((1, D_H)), _full_spec((D_H, 128))],
        out_specs=[_row_spec(D_H), _row_spec(128)],
        out_shape=[jax.ShapeDtypeStruct((NP, D_H), f32),
                   jax.ShapeDtypeStruct((NP, 128), f32)],
    )(h1h, s1, deg16, W_self1, W_neigh1, b1r, wn2p)

    s2 = _make_agg("edge")(p2p, src_p, dst_p, z128)

    outp = pl.pallas_call(
        _tc_c,
        grid=grid,
        in_specs=[_row_spec(D_H), _half_spec(128), _row_spec(16),
                  _full_spec((D_H, 128)), _full_spec((1, 128))],
        out_specs=_row_spec(128),
        out_shape=jax.ShapeDtypeStruct((NP, 128), f32),
    )(h2, s2, deg16, ws2p, b2r)

    return outp[:N, :N_CLS]


# speedup vs baseline: 1.5661x; 1.5661x over previous
import functools

import jax
import jax.numpy as jnp
from jax import lax
from jax.experimental import pallas as pl
from jax.experimental.pallas import tpu as pltpu, tpu_sc as plsc

N = 10000
E = 320000
D_IN = 128
D_H = 256
N_CLS = 47

NP = 10240
ROWS_PER = NP // 16
B = 128
EPC_E = 79 * B
EPC_C = 157 * B
EP = 32 * EPC_E


def _make_agg(mode: str):
    scratch = [
        pltpu.VMEM((B,), jnp.int32),
        pltpu.VMEM((B,), jnp.int32),
        pltpu.VMEM((B, 128), jnp.float32),
        pltpu.VMEM_SHARED((NP, 128), jnp.float32),
        pltpu.SemaphoreType.DMA,
    ]
    mesh = plsc.VectorSubcoreMesh(core_axis_name="c", subcore_axis_name="s")

    @functools.partial(
        pl.kernel, mesh=mesh,
        out_type=jax.ShapeDtypeStruct((2, NP, 128), jnp.float32),
        scratch_types=scratch)
    def agg(*refs):
        if mode == "edge":
            table, src_hbm, dst_hbm, zrows, s_out = refs[:5]
        elif mode == "chan":
            table0, table1, src_hbm, dst_hbm, zrows, s_out = refs[:6]
        else:
            dst_hbm, zrows, ones_hbm, s_out = refs[:4]
        src_v, dst_v, rows_v, acc, sem = refs[-5:]

        cid = lax.axis_index("c")
        sid = lax.axis_index("s")
        r0 = sid * ROWS_PER

        def sweep(table, e0, nchunk):
            def step(i, carry):
                off = e0 + i * B
                if mode != "deg":
                    pltpu.sync_copy(src_hbm.at[pl.ds(off, B)], src_v)
                pltpu.sync_copy(dst_hbm.at[pl.ds(off, B)], dst_v)
                if mode != "deg":
                    pltpu.async_copy(table.at[src_v], rows_v, sem).wait()
                pltpu.sync_copy(rows_v, acc.at[dst_v], add=True)
                return carry
            lax.fori_loop(0, nchunk, step, 0)

        def run(table, out_idx):
            pltpu.sync_copy(zrows.at[pl.ds(r0, ROWS_PER)],
                            acc.at[pl.ds(r0, ROWS_PER)])
            if mode == "deg":
                pltpu.sync_copy(ones_hbm, rows_v)
            plsc.subcore_barrier()
            if mode == "chan":
                sweep(table, sid * EPC_C, EPC_C // B)
            else:
                wid = sid * 2 + out_idx
                sweep(table, wid * EPC_E, EPC_E // B)
            plsc.subcore_barrier()
            pltpu.sync_copy(acc.at[pl.ds(r0, ROWS_PER)],
                            s_out.at[out_idx, pl.ds(r0, ROWS_PER)])

        if mode == "chan":
            @pl.when(cid == 0)
            def _():
                run(table0, 0)

            @pl.when(cid == 1)
            def _():
                run(table1, 1)
        else:
            tbl = None if mode == "deg" else table

            @pl.when(cid == 0)
            def _():
                run(tbl, 0)

            @pl.when(cid == 1)
            def _():
                run(tbl, 1)

    return agg


def _dot(a, b):
    return jnp.dot(a, b, preferred_element_type=jnp.float32)


BM = 512


def _inv_deg(degp_ref):
    deg = degp_ref[0][:, 0:1] + degp_ref[1][:, 0:1]
    return 1.0 / jnp.maximum(deg, 1.0)


def _tc_a(x_ref, s0_ref, degp_ref, ws_ref, wn_ref, b_ref, out_ref):
    neigh = (s0_ref[0] + s0_ref[1]) * _inv_deg(degp_ref)
    h = _dot(x_ref[...], ws_ref[...]) + _dot(neigh, wn_ref[...]) + b_ref[...]
    h = jnp.maximum(h, 0.0)
    out_ref[0] = h[:, :D_H // 2]
    out_ref[1] = h[:, D_H // 2:]


def _tc_b(h1_ref, s1_ref, degp_ref, ws_ref, wn_ref, b_ref, wn2_ref,
          h2_ref, p2_ref):
    h1 = jnp.concatenate([h1_ref[0], h1_ref[1]], axis=1)
    neigh = jnp.concatenate([s1_ref[0], s1_ref[1]], axis=1) * _inv_deg(degp_ref)
    h2 = _dot(h1, ws_ref[...]) + _dot(neigh, wn_ref[...]) + b_ref[...]
    h2 = jnp.maximum(h2, 0.0)
    h2_ref[...] = h2
    p2_ref[...] = _dot(h2, wn2_ref[...])


def _tc_c(h2_ref, s2_ref, degp_ref, ws_ref, b_ref, out_ref):
    neigh = (s2_ref[0] + s2_ref[1]) * _inv_deg(degp_ref)
    out_ref[...] = _dot(h2_ref[...], ws_ref[...]) + neigh + b_ref[...]


def _row_spec(d):
    return pl.BlockSpec((BM, d), lambda i: (i, 0))


def _half_spec(d):
    return pl.BlockSpec((2, BM, d), lambda i: (0, i, 0))


def _full_spec(shape):
    return pl.BlockSpec(shape, lambda i: tuple(0 for _ in shape))


def kernel(x, edge_index, W_self0, W_neigh0, b0, W_self1, W_neigh1, b1,
           W_self2, W_neigh2, b2):
    f32 = jnp.float32
    src = edge_index[0]
    dst = edge_index[1]
    src_p = jnp.concatenate([src, jnp.zeros((EP - E,), jnp.int32)])
    dst_p = jnp.concatenate([dst, jnp.full((EP - E,), N, jnp.int32)])

    x_p = jnp.pad(x, ((0, NP - N), (0, 0)))
    z128 = jnp.zeros((NP, 128), f32)
    ones = jnp.ones((B, 128), f32)

    wn2p = jnp.pad(W_neigh2, ((0, 0), (0, 128 - N_CLS)))
    ws2p = jnp.pad(W_self2, ((0, 0), (0, 128 - N_CLS)))
    b0r = b0.reshape(1, D_H)
    b1r = b1.reshape(1, D_H)
    b2r = jnp.pad(b2, (0, 128 - N_CLS)).reshape(1, 128)

    degp = _make_agg("deg")(dst_p, z128, ones)

    s0 = _make_agg("edge")(x_p, src_p, dst_p, z128)

    grid = (NP // BM,)
    h1h = pl.pallas_call(
        _tc_a,
        grid=grid,
        in_specs=[_row_spec(D_IN), _half_spec(128), _half_spec(128),
                  _full_spec((D_IN, D_H)), _full_spec((D_IN, D_H)),
                  _full_spec((1, D_H))],
        out_specs=_half_spec(D_H // 2),
        out_shape=jax.ShapeDtypeStruct((2, NP, D_H // 2), f32),
    )(x_p, s0, degp, W_self0, W_neigh0, b0r)

    s1 = _make_agg("chan")(h1h[0], h1h[1], src_p, dst_p, z128)

    h2, p2p = pl.pallas_call(
        _tc_b,
        grid=grid,
        in_specs=[_half_spec(D_H // 2), _half_spec(D_H // 2), _half_spec(128),
                  _full_spec((D_H, D_H)), _full_spec((D_H, D_H)),
                  _full_spec((1, D_H)), _full_spec((D_H, 128))],
        out_specs=[_row_spec(D_H), _row_spec(128)],
        out_shape=[jax.ShapeDtypeStruct((NP, D_H), f32),
                   jax.ShapeDtypeStruct((NP, 128), f32)],
    )(h1h, s1, degp, W_self1, W_neigh1, b1r, wn2p)

    s2 = _make_agg("edge")(p2p, src_p, dst_p, z128)

    outp = pl.pallas_call(
        _tc_c,
        grid=grid,
        in_specs=[_row_spec(D_H), _half_spec(128), _half_spec(128),
                  _full_spec((D_H, 128)), _full_spec((1, 128))],
        out_specs=_row_spec(128),
        out_shape=jax.ShapeDtypeStruct((NP, 128), f32),
    )(h2, s2, degp, ws2p, b2r)

    return outp[:N, :N_CLS]
